# baseline (speedup 1.0000x reference)
import jax
import jax.numpy as jnp
from jax import lax
from jax.experimental import pallas as pl
from jax.experimental.pallas import tpu as pltpu
from jax.experimental.pallas import tpu_sc as plsc

D_MODEL = 1024
BATCH = 16384
LANES = 16
NUM_CORES = 2
NUM_SUBCORES = 16
BG = BATCH // NUM_SUBCORES
DH = D_MODEL // NUM_CORES
CHUNK = 16
NRING = 4
LOOKAHEAD = 2
NUM_CHUNKS = BG // CHUNK
NGROUP = NUM_CHUNKS // NRING
NVREG = DH // LANES
REP_ROWS = NUM_CORES * NUM_SUBCORES * 64


def _sc_kernel(src_w, dst_w, mov1, mov2, out, rep, t1, t2, idx1_v, idx2_v,
               idx2t_v, b0, b1, b2, b3, g0, g2, o0, o1, o2, o3):
    bufs = (b0, b1, b2, b3)
    gsems = (g0, None, g2, None)
    osems = (o0, o1, o2, o3)
    s = lax.axis_index("s")
    c = lax.axis_index("c")
    row_base = s * BG
    col = c * DH

    pltpu.sync_copy(src_w.at[:, pl.ds(col, DH)], t1)
    pltpu.sync_copy(dst_w.at[:, pl.ds(col, DH)], t2)
    pltpu.sync_copy(mov1.at[pl.ds(row_base, BG)], idx1_v.at[pl.ds(0, BG)])
    pltpu.sync_copy(mov2.at[pl.ds(row_base, BG)], idx2_v.at[pl.ds(0, BG)])

    rep_base = (c * NUM_SUBCORES + s) * 64
    pltpu.sync_copy(t2, rep.at[pl.ds(rep_base, 64)])

    lane_block = c * (NUM_SUBCORES * 64) + lax.iota(jnp.int32, LANES) * 64

    def transform(v, carry):
        sl = pl.ds(v * LANES, LANES)
        idx2t_v[sl] = idx2_v[sl] + lane_block
        return carry

    lax.fori_loop(0, BG // LANES, transform, 0)
    plsc.subcore_barrier()

    def gather(k, buf, sem):
        off = pl.multiple_of(k * CHUNK, CHUNK)
        pltpu.async_copy(rep.at[idx2t_v.at[pl.ds(off, CHUNK)]], buf, sem)

    def wait_gather(k, buf, sem):
        off = pl.multiple_of(k * CHUNK, CHUNK)
        pltpu.make_async_copy(rep.at[idx2t_v.at[pl.ds(off, CHUNK)]], buf,
                              sem).wait()

    def out_slice(k):
        return out.at[pl.ds(row_base + k * CHUNK, CHUNK), pl.ds(col, DH)]

    def add_streamed(k, buf):
        @plsc.parallel_loop(0, CHUNK, unroll=2)
        def row_body(i):
            r1 = idx1_v[pl.ds(k * CHUNK + i, LANES)][0]
            for j in range(NVREG):
                sl = pl.ds(j * LANES, LANES)
                plsc.addupdate(buf.at[i, sl], t1[r1, sl])

    def add_local(k, buf):
        @plsc.parallel_loop(0, CHUNK, unroll=2)
        def row_body(i):
            r1 = idx1_v[pl.ds(k * CHUNK + i, LANES)][0]
            r2 = idx2_v[pl.ds(k * CHUNK + i, LANES)][0]
            for j in range(NVREG):
                sl = pl.ds(j * LANES, LANES)
                buf[i, sl] = t1[r1, sl] + t2[r2, sl]

    gather(0, bufs[0], gsems[0])

    def group_body(g, carry):
        for p in range(NRING):
            k = g * NRING + p
            if p % 2 == 0:
                kg = k + LOOKAHEAD
                q = (p + LOOKAHEAD) % NRING

                def prep(_):
                    def drain(_):
                        pltpu.make_async_copy(bufs[q],
                                              out_slice(kg - NRING),
                                              osems[q]).wait()
                        return 0

                    lax.cond(kg - NRING >= 0, drain, lambda _: 0, 0)
                    gather(kg, bufs[q], gsems[q])
                    return 0

                lax.cond(kg < NUM_CHUNKS, prep, lambda _: 0, 0)
                wait_gather(k, bufs[p], gsems[p])
                add_streamed(k, bufs[p])
            else:
                def drain(_):
                    pltpu.make_async_copy(bufs[p], out_slice(k - NRING),
                                          osems[p]).wait()
                    return 0

                lax.cond(k - NRING >= 0, drain, lambda _: 0, 0)
                add_local(k, bufs[p])
            pltpu.async_copy(bufs[p], out_slice(k), osems[p])
        return carry

    lax.fori_loop(0, NGROUP, group_body, 0)
    for p in range(NRING):
        k = NUM_CHUNKS - NRING + p
        pltpu.make_async_copy(bufs[p], out_slice(k), osems[p]).wait()


@jax.jit
def _run(src_w, dst_w, mov1, mov2):
    kern = pl.kernel(
        _sc_kernel,
        mesh=plsc.VectorSubcoreMesh(core_axis_name="c", subcore_axis_name="s"),
        out_type=jax.ShapeDtypeStruct((BATCH, D_MODEL), jnp.float32),
        scratch_types=[
            pltpu.HBM((REP_ROWS, DH), jnp.float32),
            pltpu.VMEM((64, DH), jnp.float32),
            pltpu.VMEM((64, DH), jnp.float32),
            pltpu.VMEM((BG + LANES,), jnp.int32),
            pltpu.VMEM((BG + LANES,), jnp.int32),
            pltpu.VMEM((BG,), jnp.int32),
            pltpu.VMEM((CHUNK, DH), jnp.float32),
            pltpu.VMEM((CHUNK, DH), jnp.float32),
            pltpu.VMEM((CHUNK, DH), jnp.float32),
            pltpu.VMEM((CHUNK, DH), jnp.float32),
            pltpu.SemaphoreType.DMA,
            pltpu.SemaphoreType.DMA,
            pltpu.SemaphoreType.DMA,
            pltpu.SemaphoreType.DMA,
            pltpu.SemaphoreType.DMA,
            pltpu.SemaphoreType.DMA,
        ],
    )
    return kern(src_w, dst_w, mov1, mov2)


def kernel(pieces, mov1, mov2, mov_src_w, mov_dst_w):
    del pieces
    return _run(mov_src_w, mov_dst_w, mov1, mov2)
